# Initial kernel scaffold; baseline (speedup 1.0000x reference)
import jax
import jax.numpy as jnp
from jax.experimental import pallas as pl


def kernel(xA_x, xB_x, s, is_bond_A, is_bond_B, params, Z, edge_index):
    raise NotImplementedError("write your pallas kernel here")


# speedup vs baseline: 1.2033x; 1.2033x over previous
import functools

import jax
import jax.numpy as jnp
from jax.experimental import pallas as pl
from jax.experimental.pallas import tpu as pltpu
from jax.experimental.pallas import tpu_sc as plsc

N = 10000
E = 160000
NZ = 16
HA = 64
HB = 64
AI = 64
NF = 8
SFEAT = 2 * NF + 1
SS = HA + HB + AI + SFEAT
SSP = 256
NRBF = 10
DC = 5.0
NEF = 3 * NRBF + 7
MSG = 128
HID = 128
NL = 2

BN = 2000
BE = 2000
RW = 0.5
F32 = jnp.float32


def _dot(a, b):
    return jnp.dot(a, b, preferred_element_type=F32)


def _gelu(x):
    return jax.nn.gelu(x)


def _pad_rows(w, rows):
    return jnp.concatenate([w, jnp.zeros((rows - w.shape[0], w.shape[1]), F32)], axis=0)


def _pad_cols(w, cols):
    return jnp.concatenate([w, jnp.zeros((w.shape[0], cols - w.shape[1]), F32)], axis=1)


def _c16():
    return jax.lax.broadcasted_iota(jnp.int32, (1, 16), 1).astype(F32) * (DC / (NRBF - 1))


def _rbf16(dist):
    d16 = dist * jnp.ones((1, 16), F32)
    return jnp.exp(-((d16 - _c16()) ** 2) / (2.0 * RW * RW))


def _node_pre_body(z_ref, xa_ref, xb_ref, s_ref,
                   wat0_ref, bat0_ref, wat1_ref, bat1_ref,
                   waz_ref, wax_ref, ba_ref, wbz_ref, wbx_ref, bb_ref,
                   wsrc_ref, bsrc_ref, wdst_ref,
                   h_ref, x16_ref, ps_ref, pd_ref):
    z = z_ref[...]
    zoh = (z == jax.lax.broadcasted_iota(jnp.int32, (1, NZ), 1)).astype(F32)
    atom = _dot(_gelu(_dot(zoh, wat0_ref[...]) + bat0_ref[...]), wat1_ref[...]) + bat1_ref[...]
    xa = xa_ref[...]
    xb = xb_ref[...]
    ha = jnp.tanh(_dot(zoh, waz_ref[...]) + _dot(xa, wax_ref[...]) + ba_ref[...])
    hb = jnp.tanh(_dot(zoh, wbz_ref[...]) + _dot(xb, wbx_ref[...]) + bb_ref[...])
    s = s_ref[:, :1]
    k = (jax.lax.broadcasted_iota(jnp.int32, (1, NF), 1) + 1).astype(F32)
    ang = (jnp.pi * s) * k
    h = jnp.concatenate(
        [atom, ha, hb, s, jnp.sin(ang), jnp.cos(ang),
         jnp.zeros((z.shape[0], SSP - SS), F32)], axis=1)
    h_ref[...] = h
    x16_ref[...] = (1.0 - s) * xa + s * xb
    ps_ref[...] = _dot(h, wsrc_ref[...]) + bsrc_ref[...]
    pd_ref[...] = _dot(h, wdst_ref[...])


def _node_pre(z2, xa16, xb16, s8, wat0, bat0, wat1, bat1,
              waz, wax, ba, wbz, wbx, bb, wsrc, bsrc, wdst):
    nb = N // BN
    full = lambda shape: pl.BlockSpec(shape, lambda i: (0, 0))
    row = lambda w: pl.BlockSpec((BN, w), lambda i: (i, 0))
    return pl.pallas_call(
        _node_pre_body,
        grid=(nb,),
        in_specs=[row(1), row(16), row(16), row(8),
                  full((NZ, 64)), full((1, 64)), full((64, 64)), full((1, 64)),
                  full((NZ, 64)), full((16, 64)), full((1, 64)),
                  full((NZ, 64)), full((16, 64)), full((1, 64)),
                  full((SSP, HID)), full((1, HID)), full((SSP, HID))],
        out_specs=[row(SSP), row(16), row(HID), row(HID)],
        out_shape=[jax.ShapeDtypeStruct((N, SSP), F32),
                   jax.ShapeDtypeStruct((N, 16), F32),
                   jax.ShapeDtypeStruct((N, HID), F32),
                   jax.ShapeDtypeStruct((N, HID), F32)],
    )(z2, xa16, xb16, s8, wat0, bat0, wat1, bat1, waz, wax, ba, wbz, wbx, bb,
      wsrc, bsrc, wdst)


def _edge_static_body(xas_ref, xad_ref, xbs_ref, xbd_ref, ib_ref, st_ref):
    dxa = xas_ref[...] - xad_ref[...]
    dxb = xbs_ref[...] - xbd_ref[...]
    da = jnp.sqrt(jnp.sum(dxa * dxa, axis=1, keepdims=True) + 1e-12)
    db = jnp.sqrt(jnp.sum(dxb * dxb, axis=1, keepdims=True) + 1e-12)
    st_ref[...] = jnp.concatenate(
        [ib_ref[:, :2], da, db, da - db, jnp.zeros((da.shape[0], 3), F32),
         _rbf16(da), _rbf16(db)], axis=1)


def _edge_static(xas, xad, xbs, xbd, ib2):
    nb = E // BE
    row = lambda w: pl.BlockSpec((BE, w), lambda i: (i, 0))
    return pl.pallas_call(
        _edge_static_body,
        grid=(nb,),
        in_specs=[row(16), row(16), row(16), row(16), row(8)],
        out_specs=row(40),
        out_shape=jax.ShapeDtypeStruct((E, 40), F32),
    )(xas, xad, xbs, xbd, ib2)


def _edge_msg_body(ps_ref, pd_ref, xs_ref, xd_ref, st_ref,
                   ws40_ref, wr16_ref, wd_ref, wd2_ref,
                   w2_ref, b2_ref, w3_ref, b3_ref, out_ref):
    dx = xs_ref[...] - xd_ref[...]
    dist2 = jnp.sum(dx * dx, axis=1, keepdims=True) + 1e-12
    dist = jnp.sqrt(dist2)
    pre = (ps_ref[...] + pd_ref[...]
           + _dot(st_ref[...], ws40_ref[...])
           + _dot(_rbf16(dist), wr16_ref[...])
           + dist * wd_ref[...] + dist2 * wd2_ref[...])
    m1 = _gelu(pre)
    m2 = _gelu(_dot(m1, w2_ref[...]) + b2_ref[...])
    out_ref[...] = _dot(m2, w3_ref[...]) + b3_ref[...]


def _edge_msg(psg, pdg, xs, xd, st40, ws40, wr16, wd, wd2, w2, b2, w3, b3):
    nb = E // BE
    full = lambda shape: pl.BlockSpec(shape, lambda i: (0, 0))
    row = lambda w: pl.BlockSpec((BE, w), lambda i: (i, 0))
    return pl.pallas_call(
        _edge_msg_body,
        grid=(nb,),
        in_specs=[row(HID), row(HID), row(16), row(16), row(40),
                  full((40, HID)), full((16, HID)), full((1, HID)), full((1, HID)),
                  full((HID, HID)), full((1, HID)), full((HID, MSG)), full((1, MSG))],
        out_specs=row(MSG),
        out_shape=jax.ShapeDtypeStruct((E, MSG), F32),
    )(psg, pdg, xs, xd, st40, ws40, wr16, wd, wd2, w2, b2, w3, b3)


def _edge_alpha_body(ps_ref, pd_ref, xs_ref, xd_ref, st_ref,
                     ws40_ref, wr16_ref, wd_ref, wd2_ref,
                     w2_ref, b2_ref, w3_ref, b3_ref, out_ref):
    dx = xs_ref[...] - xd_ref[...]
    dist2 = jnp.sum(dx * dx, axis=1, keepdims=True) + 1e-12
    dist = jnp.sqrt(dist2)
    pre = (ps_ref[...] + pd_ref[...]
           + _dot(st_ref[...], ws40_ref[...])
           + _dot(_rbf16(dist), wr16_ref[...])
           + dist * wd_ref[...] + dist2 * wd2_ref[...])
    a1 = _gelu(pre)
    a2 = _gelu(_dot(a1, w2_ref[...]) + b2_ref[...])
    alpha = (_dot(a2, w3_ref[...]) + b3_ref[...])[:, :1]
    out_ref[...] = alpha * dx


def _edge_alpha(psg, pdg, xs, xd, st40, ws40, wr16, wd, wd2, w2, b2, w3, b3):
    nb = E // BE
    full = lambda shape: pl.BlockSpec(shape, lambda i: (0, 0))
    row = lambda w: pl.BlockSpec((BE, w), lambda i: (i, 0))
    return pl.pallas_call(
        _edge_alpha_body,
        grid=(nb,),
        in_specs=[row(HID), row(HID), row(16), row(16), row(40),
                  full((40, HID)), full((16, HID)), full((1, HID)), full((1, HID)),
                  full((HID, HID)), full((1, HID)), full((HID, 8)), full((1, 8))],
        out_specs=row(16),
        out_shape=jax.ShapeDtypeStruct((E, 16), F32),
    )(psg, pdg, xs, xd, st40, ws40, wr16, wd, wd2, w2, b2, w3, b3)


def _node_state_body(h_ref, nm0_ref, nm1_ref, s_ref,
                     wsh_ref, wsm_ref, b1_ref, w2_ref, b2_ref, w3_ref, b3_ref,
                     was_ref, bas_ref, wad_ref,
                     wb1_ref, bb1_ref, wb2_ref, bb2_ref, wb3_ref, bb3_ref,
                     wg1_ref, bg1_ref, wg2_ref, bg2_ref, wg3_ref, bg3_ref,
                     wns_ref, bns_ref, wnd_ref,
                     h_out, qs_out, qd_out, bg_out, psn_out, pdn_out,
                     *, last):
    h = h_ref[...]
    nm = nm0_ref[...] + nm1_ref[...]
    u1 = _gelu(_dot(h, wsh_ref[...]) + _dot(nm, wsm_ref[...]) + b1_ref[...])
    u2 = _gelu(_dot(u1, w2_ref[...]) + b2_ref[...])
    hn = h + _dot(u2, w3_ref[...]) + b3_ref[...]
    h_out[...] = hn
    qs_out[...] = _dot(hn, was_ref[...]) + bas_ref[...]
    qd_out[...] = _dot(hn, wad_ref[...])
    bh1 = _gelu(_dot(hn, wb1_ref[...]) + bb1_ref[...])
    bh2 = _gelu(_dot(bh1, wb2_ref[...]) + bb2_ref[...])
    beta8 = _dot(bh2, wb3_ref[...]) + bb3_ref[...]
    gh1 = _gelu(_dot(hn, wg1_ref[...]) + bg1_ref[...])
    gh2 = _gelu(_dot(gh1, wg2_ref[...]) + bg2_ref[...])
    gamma8 = _dot(gh2, wg3_ref[...]) + bg3_ref[...]
    s = s_ref[:, :1]
    bg_out[...] = beta8 * (1.0 - s) + gamma8 * s
    if not last:
        psn_out[...] = _dot(hn, wns_ref[...]) + bns_ref[...]
        pdn_out[...] = _dot(hn, wnd_ref[...])
    else:
        psn_out[...] = jnp.zeros_like(psn_out)
        pdn_out[...] = jnp.zeros_like(pdn_out)


def _node_state(h, nm0, nm1, s8, weights, last):
    nb = N // BN
    full = lambda shape: pl.BlockSpec(shape, lambda i: (0, 0))
    row = lambda w: pl.BlockSpec((BN, w), lambda i: (i, 0))
    (wsh, wsm, b1, w2, b2, w3, b3, was, bas, wad,
     wb1, bb1, wb2, bb2, wb3, bb3, wg1, bg1, wg2, bg2, wg3, bg3,
     wns, bns, wnd) = weights
    return pl.pallas_call(
        functools.partial(_node_state_body, last=last),
        grid=(nb,),
        in_specs=[row(SSP), row(HID), row(HID), row(8),
                  full((SSP, HID)), full((HID, HID)), full((1, HID)),
                  full((HID, HID)), full((1, HID)), full((HID, SSP)), full((1, SSP)),
                  full((SSP, HID)), full((1, HID)), full((SSP, HID)),
                  full((SSP, HID)), full((1, HID)), full((HID, HID)), full((1, HID)),
                  full((HID, 8)), full((1, 8)),
                  full((SSP, HID)), full((1, HID)), full((HID, HID)), full((1, HID)),
                  full((HID, 8)), full((1, 8)),
                  full((SSP, HID)), full((1, HID)), full((SSP, HID))],
        out_specs=[row(SSP), row(HID), row(HID), row(8), row(HID), row(HID)],
        out_shape=[jax.ShapeDtypeStruct((N, SSP), F32),
                   jax.ShapeDtypeStruct((N, HID), F32),
                   jax.ShapeDtypeStruct((N, HID), F32),
                   jax.ShapeDtypeStruct((N, 8), F32),
                   jax.ShapeDtypeStruct((N, HID), F32),
                   jax.ShapeDtypeStruct((N, HID), F32)],
    )(h, nm0, nm1, s8, wsh, wsm, b1, w2, b2, w3, b3, was, bas, wad,
      wb1, bb1, wb2, bb2, wb3, bb3, wg1, bg1, wg2, bg2, wg3, bg3,
      wns, bns, wnd)


def _x_update_body(x_ref, nu0_ref, nu1_ref, bg_ref, xa_ref, xb_ref, out_ref,
                   *, last):
    x = x_ref[...]
    b = bg_ref[:, :1]
    g = bg_ref[:, 1:2]
    xn = (x + nu0_ref[...] + nu1_ref[...]
          + b * (xa_ref[...] - x) + g * (xb_ref[...] - x))
    if last:
        xn = xn - jnp.mean(xn, axis=0, keepdims=True)
    out_ref[...] = xn


def _x_update(x16, nu0, nu1, bg8, xa16, xb16, last):
    return pl.pallas_call(
        functools.partial(_x_update_body, last=last),
        out_shape=jax.ShapeDtypeStruct((N, 16), F32),
    )(x16, nu0, nu1, bg8, xa16, xb16)


def _gather4(t0, i0, t1, i1, t2, i2, t3, i3):
    return (jnp.take(t0, i0, axis=0), jnp.take(t1, i1, axis=0),
            jnp.take(t2, i2, axis=0), jnp.take(t3, i3, axis=0))


def _gather2(t0, i0, t1, i1):
    return jnp.take(t0, i0, axis=0), jnp.take(t1, i1, axis=0)


def _scatter_planes(vals, dst, width):
    z = jnp.zeros((2, N, width), F32)
    return z.at[0, dst].add(vals)


def _split_edge_w1(w1):
    wef = w1[2 * SS:]
    ws40 = jnp.concatenate(
        [wef[0:2], wef[4:7], jnp.zeros((3, HID), F32),
         wef[17:27], jnp.zeros((6, HID), F32),
         wef[27:37], jnp.zeros((6, HID), F32)], axis=0)
    wr16 = jnp.concatenate([wef[7:17], jnp.zeros((6, HID), F32)], axis=0)
    return (_pad_rows(w1[:SS], SSP), _pad_rows(w1[SS:2 * SS], SSP),
            ws40, wr16, wef[2:3], wef[3:4])


def kernel(xA_x, xB_x, s, is_bond_A, is_bond_B, params, Z, edge_index):
    s = s.reshape(-1)
    src = edge_index[0].astype(jnp.int32)
    dst = edge_index[1].astype(jnp.int32)

    z2 = Z.astype(jnp.int32).reshape(N, 1)
    xa16 = _pad_cols(xA_x, 16)
    xb16 = _pad_cols(xB_x, 16)
    s8 = _pad_cols(s.reshape(N, 1), 8)
    ib2 = _pad_cols(jnp.stack([is_bond_A, is_bond_B], axis=1), 8)

    p = params
    (wat0, bat0), (wat1, bat1) = p["atom_info"]
    (wa, ba), = p["WA"]
    (wb, bb), = p["WB"]
    waz, wax = wa[:NZ], _pad_rows(wa[NZ:], 16)
    wbz, wbx = wb[:NZ], _pad_rows(wb[NZ:], 16)

    msg_w = []
    for l in range(NL):
        (w1, b1), (w2, b2), (w3, b3) = p["msg"][l]
        wsrc, wdst, ws40, wr16, wd, wd2 = _split_edge_w1(w1)
        msg_w.append((wsrc, b1.reshape(1, -1), wdst, ws40, wr16, wd, wd2,
                      w2, b2.reshape(1, -1), w3, b3.reshape(1, -1)))
    alpha_w = []
    for l in range(NL):
        (w1, b1), (w2, b2), (w3, b3) = p["alpha"][l]
        wsrc, wdst, ws40, wr16, wd, wd2 = _split_edge_w1(w1)
        alpha_w.append((wsrc, b1.reshape(1, -1), wdst, ws40, wr16, wd, wd2,
                        w2, b2.reshape(1, -1), _pad_cols(w3, 8),
                        _pad_cols(b3.reshape(1, 1), 8)))
    node_w = []
    for l in range(NL):
        (sw1, sb1), (sw2, sb2), (sw3, sb3) = p["state"][l]
        (bw1, bb1), (bw2, bb2), (bw3, bb3) = p["beta"][l]
        (gw1, gb1), (gw2, gb2), (gw3, gb3) = p["gamma"][l]
        if l + 1 < NL:
            wns, bns, wnd = msg_w[l + 1][0], msg_w[l + 1][1], msg_w[l + 1][2]
        else:
            wns = jnp.zeros((SSP, HID), F32)
            bns = jnp.zeros((1, HID), F32)
            wnd = jnp.zeros((SSP, HID), F32)
        aws, ab1, awd = alpha_w[l][0], alpha_w[l][1], alpha_w[l][2]
        gw3p = jnp.concatenate(
            [jnp.zeros((HID, 1), F32), gw3, jnp.zeros((HID, 6), F32)], axis=1)
        gb3p = jnp.concatenate(
            [jnp.zeros((1, 1), F32), gb3.reshape(1, 1), jnp.zeros((1, 6), F32)], axis=1)
        node_w.append((
            _pad_rows(sw1[:SS], SSP), sw1[SS:], sb1.reshape(1, -1),
            sw2, sb2.reshape(1, -1), _pad_cols(sw3, SSP),
            _pad_cols(sb3.reshape(1, -1), SSP),
            aws, ab1, awd,
            _pad_rows(bw1, SSP), bb1.reshape(1, -1), bw2, bb2.reshape(1, -1),
            _pad_cols(bw3, 8), _pad_cols(bb3.reshape(1, 1), 8),
            _pad_rows(gw1, SSP), gb1.reshape(1, -1), gw2, gb2.reshape(1, -1),
            gw3p, gb3p,
            wns, bns, wnd))

    h, x16, ps, pd = _node_pre(
        z2, xa16, xb16, s8, wat0, bat0.reshape(1, -1), wat1, bat1.reshape(1, -1),
        waz, wax, ba.reshape(1, -1), wbz, wbx, bb.reshape(1, -1),
        msg_w[0][0], msg_w[0][1], msg_w[0][2])

    xas, xad, xbs, xbd = _gather4(xa16, src, xa16, dst, xb16, src, xb16, dst)
    st40 = _edge_static(xas, xad, xbs, xbd, ib2)

    for l in range(NL):
        last = (l + 1 == NL)
        mw = msg_w[l]
        aw = alpha_w[l]
        psg, pdg, xs, xd = _gather4(ps, src, pd, dst, x16, src, x16, dst)
        messages = _edge_msg(psg, pdg, xs, xd, st40,
                             mw[3], mw[4], mw[5], mw[6], mw[7], mw[8], mw[9], mw[10])
        nm = _scatter_planes(messages, dst, MSG)
        h, qs, qd, bg8, ps, pd = _node_state(h, nm[0], nm[1], s8, node_w[l], last)
        qsg, qdg = _gather2(qs, src, qd, dst)
        av = _edge_alpha(qsg, qdg, xs, xd, st40,
                         aw[3], aw[4], aw[5], aw[6], aw[7], aw[8], aw[9], aw[10])
        nu = _scatter_planes(av, dst, 16)
        x16 = _x_update(x16, nu[0], nu[1], bg8, xa16, xb16, last)

    return x16[:, :3]


# speedup vs baseline: 3.7754x; 3.1376x over previous
import functools

import jax
import jax.numpy as jnp
from jax.experimental import pallas as pl
from jax.experimental.pallas import tpu as pltpu
from jax.experimental.pallas import tpu_sc as plsc

N = 10000
E = 160000
NZ = 16
HA = 64
HB = 64
AI = 64
NF = 8
SFEAT = 2 * NF + 1
SS = HA + HB + AI + SFEAT
SSP = 256
NRBF = 10
DC = 5.0
NEF = 3 * NRBF + 7
MSG = 128
HID = 128
NL = 2

BN = 2000
BE = 2000
RW = 0.5
F32 = jnp.float32


def _dot(a, b):
    return jnp.dot(a, b, preferred_element_type=F32)


def _gelu(x):
    return jax.nn.gelu(x)


def _pad_rows(w, rows):
    return jnp.concatenate([w, jnp.zeros((rows - w.shape[0], w.shape[1]), F32)], axis=0)


def _pad_cols(w, cols):
    return jnp.concatenate([w, jnp.zeros((w.shape[0], cols - w.shape[1]), F32)], axis=1)


def _c16():
    return jax.lax.broadcasted_iota(jnp.int32, (1, 16), 1).astype(F32) * (DC / (NRBF - 1))


def _rbf16(dist):
    d16 = dist * jnp.ones((1, 16), F32)
    return jnp.exp(-((d16 - _c16()) ** 2) / (2.0 * RW * RW))


def _node_pre_body(z_ref, xa_ref, xb_ref, s_ref,
                   wat0_ref, bat0_ref, wat1_ref, bat1_ref,
                   waz_ref, wax_ref, ba_ref, wbz_ref, wbx_ref, bb_ref,
                   wsrc_ref, bsrc_ref, wdst_ref,
                   h_ref, x16_ref, ps_ref, pd_ref):
    z = z_ref[...]
    zoh = (z == jax.lax.broadcasted_iota(jnp.int32, (1, NZ), 1)).astype(F32)
    atom = _dot(_gelu(_dot(zoh, wat0_ref[...]) + bat0_ref[...]), wat1_ref[...]) + bat1_ref[...]
    xa = xa_ref[...]
    xb = xb_ref[...]
    ha = jnp.tanh(_dot(zoh, waz_ref[...]) + _dot(xa, wax_ref[...]) + ba_ref[...])
    hb = jnp.tanh(_dot(zoh, wbz_ref[...]) + _dot(xb, wbx_ref[...]) + bb_ref[...])
    s = s_ref[:, :1]
    k = (jax.lax.broadcasted_iota(jnp.int32, (1, NF), 1) + 1).astype(F32)
    ang = (jnp.pi * s) * k
    h = jnp.concatenate(
        [atom, ha, hb, s, jnp.sin(ang), jnp.cos(ang),
         jnp.zeros((z.shape[0], SSP - SS), F32)], axis=1)
    h_ref[...] = h
    x16_ref[...] = (1.0 - s) * xa + s * xb
    ps_ref[...] = _dot(h, wsrc_ref[...]) + bsrc_ref[...]
    pd_ref[...] = _dot(h, wdst_ref[...])


def _node_pre(z2, xa16, xb16, s8, wat0, bat0, wat1, bat1,
              waz, wax, ba, wbz, wbx, bb, wsrc, bsrc, wdst):
    nb = N // BN
    full = lambda shape: pl.BlockSpec(shape, lambda i: (0, 0))
    row = lambda w: pl.BlockSpec((BN, w), lambda i: (i, 0))
    return pl.pallas_call(
        _node_pre_body,
        grid=(nb,),
        in_specs=[row(1), row(16), row(16), row(8),
                  full((NZ, 64)), full((1, 64)), full((64, 64)), full((1, 64)),
                  full((NZ, 64)), full((16, 64)), full((1, 64)),
                  full((NZ, 64)), full((16, 64)), full((1, 64)),
                  full((SSP, HID)), full((1, HID)), full((SSP, HID))],
        out_specs=[row(SSP), row(16), row(HID), row(HID)],
        out_shape=[jax.ShapeDtypeStruct((N, SSP), F32),
                   jax.ShapeDtypeStruct((N, 16), F32),
                   jax.ShapeDtypeStruct((N, HID), F32),
                   jax.ShapeDtypeStruct((N, HID), F32)],
    )(z2, xa16, xb16, s8, wat0, bat0, wat1, bat1, waz, wax, ba, wbz, wbx, bb,
      wsrc, bsrc, wdst)


def _edge_static_body(xas_ref, xad_ref, xbs_ref, xbd_ref, ib_ref, st_ref):
    dxa = xas_ref[...] - xad_ref[...]
    dxb = xbs_ref[...] - xbd_ref[...]
    da = jnp.sqrt(jnp.sum(dxa * dxa, axis=1, keepdims=True) + 1e-12)
    db = jnp.sqrt(jnp.sum(dxb * dxb, axis=1, keepdims=True) + 1e-12)
    st_ref[...] = jnp.concatenate(
        [ib_ref[:, :2], da, db, da - db, jnp.zeros((da.shape[0], 3), F32),
         _rbf16(da), _rbf16(db)], axis=1)


def _edge_static(xas, xad, xbs, xbd, ib2):
    nb = E // BE
    row = lambda w: pl.BlockSpec((BE, w), lambda i: (i, 0))
    return pl.pallas_call(
        _edge_static_body,
        grid=(nb,),
        in_specs=[row(16), row(16), row(16), row(16), row(8)],
        out_specs=row(40),
        out_shape=jax.ShapeDtypeStruct((E, 40), F32),
    )(xas, xad, xbs, xbd, ib2)


def _edge_msg_body(ps_ref, pd_ref, xs_ref, xd_ref, st_ref,
                   ws40_ref, wr16_ref, wd_ref, wd2_ref,
                   w2_ref, b2_ref, w3_ref, b3_ref, out_ref):
    dx = xs_ref[...] - xd_ref[...]
    dist2 = jnp.sum(dx * dx, axis=1, keepdims=True) + 1e-12
    dist = jnp.sqrt(dist2)
    pre = (ps_ref[...] + pd_ref[...]
           + _dot(st_ref[...], ws40_ref[...])
           + _dot(_rbf16(dist), wr16_ref[...])
           + dist * wd_ref[...] + dist2 * wd2_ref[...])
    m1 = _gelu(pre)
    m2 = _gelu(_dot(m1, w2_ref[...]) + b2_ref[...])
    out_ref[...] = _dot(m2, w3_ref[...]) + b3_ref[...]


def _edge_msg(psg, pdg, xs, xd, st40, ws40, wr16, wd, wd2, w2, b2, w3, b3):
    nb = E // BE
    full = lambda shape: pl.BlockSpec(shape, lambda i: (0, 0))
    row = lambda w: pl.BlockSpec((BE, w), lambda i: (i, 0))
    return pl.pallas_call(
        _edge_msg_body,
        grid=(nb,),
        in_specs=[row(HID), row(HID), row(16), row(16), row(40),
                  full((40, HID)), full((16, HID)), full((1, HID)), full((1, HID)),
                  full((HID, HID)), full((1, HID)), full((HID, MSG)), full((1, MSG))],
        out_specs=row(MSG),
        out_shape=jax.ShapeDtypeStruct((E, MSG), F32),
    )(psg, pdg, xs, xd, st40, ws40, wr16, wd, wd2, w2, b2, w3, b3)


def _edge_alpha_body(ps_ref, pd_ref, xs_ref, xd_ref, st_ref,
                     ws40_ref, wr16_ref, wd_ref, wd2_ref,
                     w2_ref, b2_ref, w3_ref, b3_ref, out_ref):
    dx = xs_ref[...] - xd_ref[...]
    dist2 = jnp.sum(dx * dx, axis=1, keepdims=True) + 1e-12
    dist = jnp.sqrt(dist2)
    pre = (ps_ref[...] + pd_ref[...]
           + _dot(st_ref[...], ws40_ref[...])
           + _dot(_rbf16(dist), wr16_ref[...])
           + dist * wd_ref[...] + dist2 * wd2_ref[...])
    a1 = _gelu(pre)
    a2 = _gelu(_dot(a1, w2_ref[...]) + b2_ref[...])
    alpha = (_dot(a2, w3_ref[...]) + b3_ref[...])[:, :1]
    out_ref[...] = alpha * dx


def _edge_alpha(psg, pdg, xs, xd, st40, ws40, wr16, wd, wd2, w2, b2, w3, b3):
    nb = E // BE
    full = lambda shape: pl.BlockSpec(shape, lambda i: (0, 0))
    row = lambda w: pl.BlockSpec((BE, w), lambda i: (i, 0))
    return pl.pallas_call(
        _edge_alpha_body,
        grid=(nb,),
        in_specs=[row(HID), row(HID), row(16), row(16), row(40),
                  full((40, HID)), full((16, HID)), full((1, HID)), full((1, HID)),
                  full((HID, HID)), full((1, HID)), full((HID, 8)), full((1, 8))],
        out_specs=row(16),
        out_shape=jax.ShapeDtypeStruct((E, 16), F32),
    )(psg, pdg, xs, xd, st40, ws40, wr16, wd, wd2, w2, b2, w3, b3)


def _node_state_body(h_ref, nm0_ref, nm1_ref, s_ref,
                     wsh_ref, wsm_ref, b1_ref, w2_ref, b2_ref, w3_ref, b3_ref,
                     was_ref, bas_ref, wad_ref,
                     wb1_ref, bb1_ref, wb2_ref, bb2_ref, wb3_ref, bb3_ref,
                     wg1_ref, bg1_ref, wg2_ref, bg2_ref, wg3_ref, bg3_ref,
                     wns_ref, bns_ref, wnd_ref,
                     h_out, qs_out, qd_out, bg_out, psn_out, pdn_out,
                     *, last):
    h = h_ref[...]
    nm = nm0_ref[...] + nm1_ref[...]
    u1 = _gelu(_dot(h, wsh_ref[...]) + _dot(nm, wsm_ref[...]) + b1_ref[...])
    u2 = _gelu(_dot(u1, w2_ref[...]) + b2_ref[...])
    hn = h + _dot(u2, w3_ref[...]) + b3_ref[...]
    h_out[...] = hn
    qs_out[...] = _dot(hn, was_ref[...]) + bas_ref[...]
    qd_out[...] = _dot(hn, wad_ref[...])
    bh1 = _gelu(_dot(hn, wb1_ref[...]) + bb1_ref[...])
    bh2 = _gelu(_dot(bh1, wb2_ref[...]) + bb2_ref[...])
    beta8 = _dot(bh2, wb3_ref[...]) + bb3_ref[...]
    gh1 = _gelu(_dot(hn, wg1_ref[...]) + bg1_ref[...])
    gh2 = _gelu(_dot(gh1, wg2_ref[...]) + bg2_ref[...])
    gamma8 = _dot(gh2, wg3_ref[...]) + bg3_ref[...]
    s = s_ref[:, :1]
    bg_out[...] = beta8 * (1.0 - s) + gamma8 * s
    if not last:
        psn_out[...] = _dot(hn, wns_ref[...]) + bns_ref[...]
        pdn_out[...] = _dot(hn, wnd_ref[...])
    else:
        psn_out[...] = jnp.zeros_like(psn_out)
        pdn_out[...] = jnp.zeros_like(pdn_out)


def _node_state(h, nm0, nm1, s8, weights, last):
    nb = N // BN
    full = lambda shape: pl.BlockSpec(shape, lambda i: (0, 0))
    row = lambda w: pl.BlockSpec((BN, w), lambda i: (i, 0))
    (wsh, wsm, b1, w2, b2, w3, b3, was, bas, wad,
     wb1, bb1, wb2, bb2, wb3, bb3, wg1, bg1, wg2, bg2, wg3, bg3,
     wns, bns, wnd) = weights
    return pl.pallas_call(
        functools.partial(_node_state_body, last=last),
        grid=(nb,),
        in_specs=[row(SSP), row(HID), row(HID), row(8),
                  full((SSP, HID)), full((HID, HID)), full((1, HID)),
                  full((HID, HID)), full((1, HID)), full((HID, SSP)), full((1, SSP)),
                  full((SSP, HID)), full((1, HID)), full((SSP, HID)),
                  full((SSP, HID)), full((1, HID)), full((HID, HID)), full((1, HID)),
                  full((HID, 8)), full((1, 8)),
                  full((SSP, HID)), full((1, HID)), full((HID, HID)), full((1, HID)),
                  full((HID, 8)), full((1, 8)),
                  full((SSP, HID)), full((1, HID)), full((SSP, HID))],
        out_specs=[row(SSP), row(HID), row(HID), row(8), row(HID), row(HID)],
        out_shape=[jax.ShapeDtypeStruct((N, SSP), F32),
                   jax.ShapeDtypeStruct((N, HID), F32),
                   jax.ShapeDtypeStruct((N, HID), F32),
                   jax.ShapeDtypeStruct((N, 8), F32),
                   jax.ShapeDtypeStruct((N, HID), F32),
                   jax.ShapeDtypeStruct((N, HID), F32)],
    )(h, nm0, nm1, s8, wsh, wsm, b1, w2, b2, w3, b3, was, bas, wad,
      wb1, bb1, wb2, bb2, wb3, bb3, wg1, bg1, wg2, bg2, wg3, bg3,
      wns, bns, wnd)


def _x_update_body(x_ref, nu0_ref, nu1_ref, bg_ref, xa_ref, xb_ref, out_ref,
                   *, last):
    x = x_ref[...]
    b = bg_ref[:, :1]
    g = bg_ref[:, 1:2]
    xn = (x + nu0_ref[...] + nu1_ref[...]
          + b * (xa_ref[...] - x) + g * (xb_ref[...] - x))
    if last:
        xn = xn - jnp.mean(xn, axis=0, keepdims=True)
    out_ref[...] = xn


def _x_update(x16, nu0, nu1, bg8, xa16, xb16, last):
    return pl.pallas_call(
        functools.partial(_x_update_body, last=last),
        out_shape=jax.ShapeDtypeStruct((N, 16), F32),
    )(x16, nu0, nu1, bg8, xa16, xb16)


_NW = 32
_PW = E // _NW
_CH = 128
_NFULL = _PW // _CH
_TAIL = _PW - _NFULL * _CH
_SUBROWS = N // 16


def _sc_mesh():
    return plsc.VectorSubcoreMesh(core_axis_name="c", subcore_axis_name="s")


_SC_PARAMS = pltpu.CompilerParams(use_tc_tiling_on_sc=False)


def _make_gather(widths):
    k_tabs = len(widths)
    scratch = []
    for w in widths:
        scratch += [pltpu.VMEM((_CH,), jnp.int32),
                    pltpu.VMEM((_CH, w), F32),
                    pltpu.SemaphoreType.DMA]
    out_type = [jax.ShapeDtypeStruct((E, w), F32) for w in widths]

    @functools.partial(pl.kernel, out_type=out_type, mesh=_sc_mesh(),
                       scratch_types=scratch, compiler_params=_SC_PARAMS)
    def kern(*refs):
        tabs = refs[:k_tabs]
        idxs = refs[k_tabs:2 * k_tabs]
        outs = refs[2 * k_tabs:3 * k_tabs]
        scr = refs[3 * k_tabs:]
        wid = jax.lax.axis_index("s") * 2 + jax.lax.axis_index("c")
        base0 = wid * _PW

        def full_chunk(base):
            for j in range(k_tabs):
                pltpu.sync_copy(idxs[j].at[pl.ds(base, _CH)], scr[3 * j])
            cps = [pltpu.async_copy(tabs[j].at[scr[3 * j]], scr[3 * j + 1],
                                    scr[3 * j + 2])
                   for j in range(k_tabs)]
            for c in cps:
                c.wait()
            for j in range(k_tabs):
                pltpu.sync_copy(scr[3 * j + 1], outs[j].at[pl.ds(base, _CH)])

        @pl.loop(0, _NFULL)
        def _(i):
            full_chunk(base0 + i * _CH)

        if _TAIL:
            base = base0 + _NFULL * _CH
            for j in range(k_tabs):
                iv = scr[3 * j].at[pl.ds(0, _TAIL)]
                rv = scr[3 * j + 1].at[pl.ds(0, _TAIL)]
                pltpu.sync_copy(idxs[j].at[pl.ds(base, _TAIL)], iv)
                pltpu.sync_copy(tabs[j].at[iv], rv)
                pltpu.sync_copy(rv, outs[j].at[pl.ds(base, _TAIL)])

    return kern


def _make_scatter(w):
    scratch = [pltpu.VMEM((_CH,), jnp.int32),
               pltpu.VMEM((_CH, w), F32),
               pltpu.VMEM((_TAIL,), jnp.int32),
               pltpu.VMEM((_TAIL, w), F32),
               pltpu.VMEM_SHARED((N, w), F32)]

    @functools.partial(pl.kernel,
                       out_type=jax.ShapeDtypeStruct((2, N, w), F32),
                       mesh=_sc_mesh(), scratch_types=scratch,
                       compiler_params=_SC_PARAMS)
    def kern(vals, dsti, zrows, out, iv, rv, ivt, rvt, acc):
        cid = jax.lax.axis_index("c")
        sid = jax.lax.axis_index("s")
        base0 = (sid * 2 + cid) * _PW
        stripe = pl.ds(sid * _SUBROWS, _SUBROWS)
        pltpu.sync_copy(zrows.at[stripe], acc.at[stripe])
        plsc.subcore_barrier()

        @pl.loop(0, _NFULL)
        def _(i):
            base = base0 + i * _CH
            pltpu.sync_copy(dsti.at[pl.ds(base, _CH)], iv)
            pltpu.sync_copy(vals.at[pl.ds(base, _CH)], rv)
            pltpu.sync_copy(rv, acc.at[iv], add=True)

        if _TAIL:
            base = base0 + _NFULL * _CH
            pltpu.sync_copy(dsti.at[pl.ds(base, _TAIL)], ivt)
            pltpu.sync_copy(vals.at[pl.ds(base, _TAIL)], rvt)
            pltpu.sync_copy(rvt, acc.at[ivt], add=True)

        plsc.subcore_barrier()
        pltpu.sync_copy(acc.at[stripe], out.at[cid, stripe])

    return kern


_GATHER_PPXX = _make_gather((HID, HID, 16, 16))
_GATHER_XXXX = _make_gather((16, 16, 16, 16))
_GATHER_QQ = _make_gather((HID, HID))
_SCATTER_MSG = _make_scatter(MSG)
_SCATTER_POS = _make_scatter(16)


def _gather4(t0, i0, t1, i1, t2, i2, t3, i3):
    if t0.shape[1] == HID:
        return _GATHER_PPXX(t0, t1, t2, t3, i0, i1, i2, i3)
    return _GATHER_XXXX(t0, t1, t2, t3, i0, i1, i2, i3)


def _gather2(t0, i0, t1, i1):
    return _GATHER_QQ(t0, t1, i0, i1)


def _scatter_planes(vals, dst, width):
    zrows = jnp.zeros((N, width), F32)
    if width == MSG:
        return _SCATTER_MSG(vals, dst, zrows)
    return _SCATTER_POS(vals, dst, zrows)


def _split_edge_w1(w1):
    wef = w1[2 * SS:]
    ws40 = jnp.concatenate(
        [wef[0:2], wef[4:7], jnp.zeros((3, HID), F32),
         wef[17:27], jnp.zeros((6, HID), F32),
         wef[27:37], jnp.zeros((6, HID), F32)], axis=0)
    wr16 = jnp.concatenate([wef[7:17], jnp.zeros((6, HID), F32)], axis=0)
    return (_pad_rows(w1[:SS], SSP), _pad_rows(w1[SS:2 * SS], SSP),
            ws40, wr16, wef[2:3], wef[3:4])


def kernel(xA_x, xB_x, s, is_bond_A, is_bond_B, params, Z, edge_index):
    s = s.reshape(-1)
    src = edge_index[0].astype(jnp.int32)
    dst = edge_index[1].astype(jnp.int32)

    z2 = Z.astype(jnp.int32).reshape(N, 1)
    xa16 = _pad_cols(xA_x, 16)
    xb16 = _pad_cols(xB_x, 16)
    s8 = _pad_cols(s.reshape(N, 1), 8)
    ib2 = _pad_cols(jnp.stack([is_bond_A, is_bond_B], axis=1), 8)

    p = params
    (wat0, bat0), (wat1, bat1) = p["atom_info"]
    (wa, ba), = p["WA"]
    (wb, bb), = p["WB"]
    waz, wax = wa[:NZ], _pad_rows(wa[NZ:], 16)
    wbz, wbx = wb[:NZ], _pad_rows(wb[NZ:], 16)

    msg_w = []
    for l in range(NL):
        (w1, b1), (w2, b2), (w3, b3) = p["msg"][l]
        wsrc, wdst, ws40, wr16, wd, wd2 = _split_edge_w1(w1)
        msg_w.append((wsrc, b1.reshape(1, -1), wdst, ws40, wr16, wd, wd2,
                      w2, b2.reshape(1, -1), w3, b3.reshape(1, -1)))
    alpha_w = []
    for l in range(NL):
        (w1, b1), (w2, b2), (w3, b3) = p["alpha"][l]
        wsrc, wdst, ws40, wr16, wd, wd2 = _split_edge_w1(w1)
        alpha_w.append((wsrc, b1.reshape(1, -1), wdst, ws40, wr16, wd, wd2,
                        w2, b2.reshape(1, -1), _pad_cols(w3, 8),
                        _pad_cols(b3.reshape(1, 1), 8)))
    node_w = []
    for l in range(NL):
        (sw1, sb1), (sw2, sb2), (sw3, sb3) = p["state"][l]
        (bw1, bb1), (bw2, bb2), (bw3, bb3) = p["beta"][l]
        (gw1, gb1), (gw2, gb2), (gw3, gb3) = p["gamma"][l]
        if l + 1 < NL:
            wns, bns, wnd = msg_w[l + 1][0], msg_w[l + 1][1], msg_w[l + 1][2]
        else:
            wns = jnp.zeros((SSP, HID), F32)
            bns = jnp.zeros((1, HID), F32)
            wnd = jnp.zeros((SSP, HID), F32)
        aws, ab1, awd = alpha_w[l][0], alpha_w[l][1], alpha_w[l][2]
        gw3p = jnp.concatenate(
            [jnp.zeros((HID, 1), F32), gw3, jnp.zeros((HID, 6), F32)], axis=1)
        gb3p = jnp.concatenate(
            [jnp.zeros((1, 1), F32), gb3.reshape(1, 1), jnp.zeros((1, 6), F32)], axis=1)
        node_w.append((
            _pad_rows(sw1[:SS], SSP), sw1[SS:], sb1.reshape(1, -1),
            sw2, sb2.reshape(1, -1), _pad_cols(sw3, SSP),
            _pad_cols(sb3.reshape(1, -1), SSP),
            aws, ab1, awd,
            _pad_rows(bw1, SSP), bb1.reshape(1, -1), bw2, bb2.reshape(1, -1),
            _pad_cols(bw3, 8), _pad_cols(bb3.reshape(1, 1), 8),
            _pad_rows(gw1, SSP), gb1.reshape(1, -1), gw2, gb2.reshape(1, -1),
            gw3p, gb3p,
            wns, bns, wnd))

    h, x16, ps, pd = _node_pre(
        z2, xa16, xb16, s8, wat0, bat0.reshape(1, -1), wat1, bat1.reshape(1, -1),
        waz, wax, ba.reshape(1, -1), wbz, wbx, bb.reshape(1, -1),
        msg_w[0][0], msg_w[0][1], msg_w[0][2])

    xas, xad, xbs, xbd = _gather4(xa16, src, xa16, dst, xb16, src, xb16, dst)
    st40 = _edge_static(xas, xad, xbs, xbd, ib2)

    for l in range(NL):
        last = (l + 1 == NL)
        mw = msg_w[l]
        aw = alpha_w[l]
        psg, pdg, xs, xd = _gather4(ps, src, pd, dst, x16, src, x16, dst)
        messages = _edge_msg(psg, pdg, xs, xd, st40,
                             mw[3], mw[4], mw[5], mw[6], mw[7], mw[8], mw[9], mw[10])
        nm = _scatter_planes(messages, dst, MSG)
        h, qs, qd, bg8, ps, pd = _node_state(h, nm[0], nm[1], s8, node_w[l], last)
        qsg, qdg = _gather2(qs, src, qd, dst)
        av = _edge_alpha(qsg, qdg, xs, xd, st40,
                         aw[3], aw[4], aw[5], aw[6], aw[7], aw[8], aw[9], aw[10])
        nu = _scatter_planes(av, dst, 16)
        x16 = _x_update(x16, nu[0], nu[1], bg8, xa16, xb16, last)

    return x16[:, :3]


# speedup vs baseline: 4.2995x; 1.1388x over previous
import functools

import jax
import jax.numpy as jnp
from jax.experimental import pallas as pl
from jax.experimental.pallas import tpu as pltpu
from jax.experimental.pallas import tpu_sc as plsc

N = 10000
E = 160000
NZ = 16
HA = 64
HB = 64
AI = 64
NF = 8
SFEAT = 2 * NF + 1
SS = HA + HB + AI + SFEAT
SSP = 256
NRBF = 10
DC = 5.0
NEF = 3 * NRBF + 7
MSG = 128
HID = 128
NL = 2

BN = 2000
BE = 2000
RW = 0.5
F32 = jnp.float32


def _dot(a, b):
    return jnp.dot(a, b, preferred_element_type=F32)


def _gelu(x):
    return jax.nn.gelu(x)


def _pad_rows(w, rows):
    return jnp.concatenate([w, jnp.zeros((rows - w.shape[0], w.shape[1]), F32)], axis=0)


def _pad_cols(w, cols):
    return jnp.concatenate([w, jnp.zeros((w.shape[0], cols - w.shape[1]), F32)], axis=1)


def _c16():
    return jax.lax.broadcasted_iota(jnp.int32, (1, 16), 1).astype(F32) * (DC / (NRBF - 1))


def _rbf16(dist):
    d16 = dist * jnp.ones((1, 16), F32)
    return jnp.exp(-((d16 - _c16()) ** 2) / (2.0 * RW * RW))


def _node_pre_body(z_ref, xa_ref, xb_ref, s_ref,
                   wat0_ref, bat0_ref, wat1_ref, bat1_ref,
                   waz_ref, wax_ref, ba_ref, wbz_ref, wbx_ref, bb_ref,
                   wsrc_ref, bsrc_ref, wdst_ref,
                   h_ref, x16_ref, ps_ref, pd_ref):
    z = z_ref[...]
    zoh = (z == jax.lax.broadcasted_iota(jnp.int32, (1, NZ), 1)).astype(F32)
    atom = _dot(_gelu(_dot(zoh, wat0_ref[...]) + bat0_ref[...]), wat1_ref[...]) + bat1_ref[...]
    xa = xa_ref[...]
    xb = xb_ref[...]
    ha = jnp.tanh(_dot(zoh, waz_ref[...]) + _dot(xa, wax_ref[...]) + ba_ref[...])
    hb = jnp.tanh(_dot(zoh, wbz_ref[...]) + _dot(xb, wbx_ref[...]) + bb_ref[...])
    s = s_ref[:, :1]
    k = (jax.lax.broadcasted_iota(jnp.int32, (1, NF), 1) + 1).astype(F32)
    ang = (jnp.pi * s) * k
    h = jnp.concatenate(
        [atom, ha, hb, s, jnp.sin(ang), jnp.cos(ang),
         jnp.zeros((z.shape[0], SSP - SS), F32)], axis=1)
    h_ref[...] = h
    x16_ref[...] = (1.0 - s) * xa + s * xb
    ps_ref[...] = _dot(h, wsrc_ref[...]) + bsrc_ref[...]
    pd_ref[...] = _dot(h, wdst_ref[...])


def _node_pre(z2, xa16, xb16, s8, wat0, bat0, wat1, bat1,
              waz, wax, ba, wbz, wbx, bb, wsrc, bsrc, wdst):
    nb = N // BN
    full = lambda shape: pl.BlockSpec(shape, lambda i: (0, 0))
    row = lambda w: pl.BlockSpec((BN, w), lambda i: (i, 0))
    return pl.pallas_call(
        _node_pre_body,
        grid=(nb,),
        in_specs=[row(1), row(16), row(16), row(8),
                  full((NZ, 64)), full((1, 64)), full((64, 64)), full((1, 64)),
                  full((NZ, 64)), full((16, 64)), full((1, 64)),
                  full((NZ, 64)), full((16, 64)), full((1, 64)),
                  full((SSP, HID)), full((1, HID)), full((SSP, HID))],
        out_specs=[row(SSP), row(16), row(HID), row(HID)],
        out_shape=[jax.ShapeDtypeStruct((N, SSP), F32),
                   jax.ShapeDtypeStruct((N, 16), F32),
                   jax.ShapeDtypeStruct((N, HID), F32),
                   jax.ShapeDtypeStruct((N, HID), F32)],
    )(z2, xa16, xb16, s8, wat0, bat0, wat1, bat1, waz, wax, ba, wbz, wbx, bb,
      wsrc, bsrc, wdst)


def _edge_static_body(xas_ref, xad_ref, xbs_ref, xbd_ref, ib_ref, st_ref):
    dxa = xas_ref[...] - xad_ref[...]
    dxb = xbs_ref[...] - xbd_ref[...]
    da = jnp.sqrt(jnp.sum(dxa * dxa, axis=1, keepdims=True) + 1e-12)
    db = jnp.sqrt(jnp.sum(dxb * dxb, axis=1, keepdims=True) + 1e-12)
    st_ref[...] = jnp.concatenate(
        [ib_ref[:, :2], da, db, da - db, jnp.zeros((da.shape[0], 3), F32),
         _rbf16(da), _rbf16(db)], axis=1)


def _edge_static(xas, xad, xbs, xbd, ib2):
    nb = E // BE
    row = lambda w: pl.BlockSpec((BE, w), lambda i: (i, 0))
    return pl.pallas_call(
        _edge_static_body,
        grid=(nb,),
        in_specs=[row(16), row(16), row(16), row(16), row(8)],
        out_specs=row(40),
        out_shape=jax.ShapeDtypeStruct((E, 40), F32),
    )(xas, xad, xbs, xbd, ib2)


def _edge_msg_body(ps_ref, pd_ref, xs_ref, xd_ref, st_ref,
                   ws40_ref, wr16_ref, wd_ref, wd2_ref,
                   w2_ref, b2_ref, w3_ref, b3_ref, out_ref):
    dx = xs_ref[...] - xd_ref[...]
    dist2 = jnp.sum(dx * dx, axis=1, keepdims=True) + 1e-12
    dist = jnp.sqrt(dist2)
    pre = (ps_ref[...] + pd_ref[...]
           + _dot(st_ref[...], ws40_ref[...])
           + _dot(_rbf16(dist), wr16_ref[...])
           + dist * wd_ref[...] + dist2 * wd2_ref[...])
    m1 = _gelu(pre)
    m2 = _gelu(_dot(m1, w2_ref[...]) + b2_ref[...])
    out_ref[...] = _dot(m2, w3_ref[...]) + b3_ref[...]


def _edge_msg(psg, pdg, xs, xd, st40, ws40, wr16, wd, wd2, w2, b2, w3, b3):
    nb = E // BE
    full = lambda shape: pl.BlockSpec(shape, lambda i: (0, 0))
    row = lambda w: pl.BlockSpec((BE, w), lambda i: (i, 0))
    return pl.pallas_call(
        _edge_msg_body,
        grid=(nb,),
        in_specs=[row(HID), row(HID), row(16), row(16), row(40),
                  full((40, HID)), full((16, HID)), full((1, HID)), full((1, HID)),
                  full((HID, HID)), full((1, HID)), full((HID, MSG)), full((1, MSG))],
        out_specs=row(MSG),
        out_shape=jax.ShapeDtypeStruct((E, MSG), F32),
    )(psg, pdg, xs, xd, st40, ws40, wr16, wd, wd2, w2, b2, w3, b3)


def _edge_alpha_body(ps_ref, pd_ref, xs_ref, xd_ref, st_ref,
                     ws40_ref, wr16_ref, wd_ref, wd2_ref,
                     w2_ref, b2_ref, w3_ref, b3_ref, out_ref):
    dx = xs_ref[...] - xd_ref[...]
    dist2 = jnp.sum(dx * dx, axis=1, keepdims=True) + 1e-12
    dist = jnp.sqrt(dist2)
    pre = (ps_ref[...] + pd_ref[...]
           + _dot(st_ref[...], ws40_ref[...])
           + _dot(_rbf16(dist), wr16_ref[...])
           + dist * wd_ref[...] + dist2 * wd2_ref[...])
    a1 = _gelu(pre)
    a2 = _gelu(_dot(a1, w2_ref[...]) + b2_ref[...])
    alpha = (_dot(a2, w3_ref[...]) + b3_ref[...])[:, :1]
    out_ref[...] = alpha * dx


def _edge_alpha(psg, pdg, xs, xd, st40, ws40, wr16, wd, wd2, w2, b2, w3, b3):
    nb = E // BE
    full = lambda shape: pl.BlockSpec(shape, lambda i: (0, 0))
    row = lambda w: pl.BlockSpec((BE, w), lambda i: (i, 0))
    return pl.pallas_call(
        _edge_alpha_body,
        grid=(nb,),
        in_specs=[row(HID), row(HID), row(16), row(16), row(40),
                  full((40, HID)), full((16, HID)), full((1, HID)), full((1, HID)),
                  full((HID, HID)), full((1, HID)), full((HID, 8)), full((1, 8))],
        out_specs=row(16),
        out_shape=jax.ShapeDtypeStruct((E, 16), F32),
    )(psg, pdg, xs, xd, st40, ws40, wr16, wd, wd2, w2, b2, w3, b3)


def _node_state_body(h_ref, nm0_ref, nm1_ref, s_ref,
                     wsh_ref, wsm_ref, b1_ref, w2_ref, b2_ref, w3_ref, b3_ref,
                     was_ref, bas_ref, wad_ref,
                     wb1_ref, bb1_ref, wb2_ref, bb2_ref, wb3_ref, bb3_ref,
                     wg1_ref, bg1_ref, wg2_ref, bg2_ref, wg3_ref, bg3_ref,
                     wns_ref, bns_ref, wnd_ref,
                     h_out, qs_out, qd_out, bg_out, psn_out, pdn_out,
                     *, last):
    h = h_ref[...]
    nm = nm0_ref[...] + nm1_ref[...]
    u1 = _gelu(_dot(h, wsh_ref[...]) + _dot(nm, wsm_ref[...]) + b1_ref[...])
    u2 = _gelu(_dot(u1, w2_ref[...]) + b2_ref[...])
    hn = h + _dot(u2, w3_ref[...]) + b3_ref[...]
    h_out[...] = hn
    qs_out[...] = _dot(hn, was_ref[...]) + bas_ref[...]
    qd_out[...] = _dot(hn, wad_ref[...])
    bh1 = _gelu(_dot(hn, wb1_ref[...]) + bb1_ref[...])
    bh2 = _gelu(_dot(bh1, wb2_ref[...]) + bb2_ref[...])
    beta8 = _dot(bh2, wb3_ref[...]) + bb3_ref[...]
    gh1 = _gelu(_dot(hn, wg1_ref[...]) + bg1_ref[...])
    gh2 = _gelu(_dot(gh1, wg2_ref[...]) + bg2_ref[...])
    gamma8 = _dot(gh2, wg3_ref[...]) + bg3_ref[...]
    s = s_ref[:, :1]
    bg_out[...] = beta8 * (1.0 - s) + gamma8 * s
    if not last:
        psn_out[...] = _dot(hn, wns_ref[...]) + bns_ref[...]
        pdn_out[...] = _dot(hn, wnd_ref[...])
    else:
        psn_out[...] = jnp.zeros_like(psn_out)
        pdn_out[...] = jnp.zeros_like(pdn_out)


def _node_state(h, nm0, nm1, s8, weights, last):
    nb = N // BN
    full = lambda shape: pl.BlockSpec(shape, lambda i: (0, 0))
    row = lambda w: pl.BlockSpec((BN, w), lambda i: (i, 0))
    (wsh, wsm, b1, w2, b2, w3, b3, was, bas, wad,
     wb1, bb1, wb2, bb2, wb3, bb3, wg1, bg1, wg2, bg2, wg3, bg3,
     wns, bns, wnd) = weights
    return pl.pallas_call(
        functools.partial(_node_state_body, last=last),
        grid=(nb,),
        in_specs=[row(SSP), row(HID), row(HID), row(8),
                  full((SSP, HID)), full((HID, HID)), full((1, HID)),
                  full((HID, HID)), full((1, HID)), full((HID, SSP)), full((1, SSP)),
                  full((SSP, HID)), full((1, HID)), full((SSP, HID)),
                  full((SSP, HID)), full((1, HID)), full((HID, HID)), full((1, HID)),
                  full((HID, 8)), full((1, 8)),
                  full((SSP, HID)), full((1, HID)), full((HID, HID)), full((1, HID)),
                  full((HID, 8)), full((1, 8)),
                  full((SSP, HID)), full((1, HID)), full((SSP, HID))],
        out_specs=[row(SSP), row(HID), row(HID), row(8), row(HID), row(HID)],
        out_shape=[jax.ShapeDtypeStruct((N, SSP), F32),
                   jax.ShapeDtypeStruct((N, HID), F32),
                   jax.ShapeDtypeStruct((N, HID), F32),
                   jax.ShapeDtypeStruct((N, 8), F32),
                   jax.ShapeDtypeStruct((N, HID), F32),
                   jax.ShapeDtypeStruct((N, HID), F32)],
    )(h, nm0, nm1, s8, wsh, wsm, b1, w2, b2, w3, b3, was, bas, wad,
      wb1, bb1, wb2, bb2, wb3, bb3, wg1, bg1, wg2, bg2, wg3, bg3,
      wns, bns, wnd)


def _x_update_body(x_ref, nu0_ref, nu1_ref, bg_ref, xa_ref, xb_ref, out_ref,
                   *, last):
    x = x_ref[...]
    b = bg_ref[:, :1]
    g = bg_ref[:, 1:2]
    xn = (x + nu0_ref[...] + nu1_ref[...]
          + b * (xa_ref[...] - x) + g * (xb_ref[...] - x))
    if last:
        xn = xn - jnp.mean(xn, axis=0, keepdims=True)
    out_ref[...] = xn


def _x_update(x16, nu0, nu1, bg8, xa16, xb16, last):
    return pl.pallas_call(
        functools.partial(_x_update_body, last=last),
        out_shape=jax.ShapeDtypeStruct((N, 16), F32),
    )(x16, nu0, nu1, bg8, xa16, xb16)


_NW = 32
_PW = E // _NW
_CH = 128
_NFULL = _PW // _CH
_TAIL = _PW - _NFULL * _CH
_SUBROWS = N // 16


def _sc_mesh():
    return plsc.VectorSubcoreMesh(core_axis_name="c", subcore_axis_name="s")


_SC_PARAMS = pltpu.CompilerParams(use_tc_tiling_on_sc=False)


def _make_gather(widths):
    k_tabs = len(widths)
    scratch = []
    for w in widths:
        for _ in range(2):
            scratch += [pltpu.VMEM((_CH,), jnp.int32),
                        pltpu.VMEM((_CH, w), F32),
                        pltpu.SemaphoreType.DMA]
    out_type = [jax.ShapeDtypeStruct((E, w), F32) for w in widths]

    @functools.partial(pl.kernel, out_type=out_type, mesh=_sc_mesh(),
                       scratch_types=scratch, compiler_params=_SC_PARAMS)
    def kern(*refs):
        tabs = refs[:k_tabs]
        idxs = refs[k_tabs:2 * k_tabs]
        outs = refs[2 * k_tabs:3 * k_tabs]
        scr = refs[3 * k_tabs:]

        def buf(j, p):
            o = 6 * j + 3 * p
            return scr[o], scr[o + 1], scr[o + 2]

        wid = jax.lax.axis_index("s") * 2 + jax.lax.axis_index("c")
        base0 = wid * _PW

        def start(c, p):
            base = base0 + c * _CH
            for j in range(k_tabs):
                iv, rv, sem = buf(j, p)
                pltpu.sync_copy(idxs[j].at[pl.ds(base, _CH)], iv)
                pltpu.async_copy(tabs[j].at[iv], rv, sem)

        def finish(c, p):
            base = base0 + c * _CH
            for j in range(k_tabs):
                iv, rv, sem = buf(j, p)
                pltpu.make_async_copy(tabs[j].at[iv], rv, sem).wait()
            for j in range(k_tabs):
                _, rv, _ = buf(j, p)
                pltpu.sync_copy(rv, outs[j].at[pl.ds(base, _CH)])

        start(0, 0)
        if _NFULL > 1:
            start(1, 1)

        @pl.loop(0, (_NFULL + 1) // 2)
        def _(i):
            c0 = 2 * i
            finish(c0, 0)

            @pl.when(c0 + 2 < _NFULL)
            def _():
                start(c0 + 2, 0)

            @pl.when(c0 + 1 < _NFULL)
            def _():
                finish(c0 + 1, 1)

            @pl.when(c0 + 3 < _NFULL)
            def _():
                start(c0 + 3, 1)

        if _TAIL:
            base = base0 + _NFULL * _CH
            for j in range(k_tabs):
                iv0, rv0, _ = buf(j, 0)
                iv = iv0.at[pl.ds(0, _TAIL)]
                rv = rv0.at[pl.ds(0, _TAIL)]
                pltpu.sync_copy(idxs[j].at[pl.ds(base, _TAIL)], iv)
                pltpu.sync_copy(tabs[j].at[iv], rv)
                pltpu.sync_copy(rv, outs[j].at[pl.ds(base, _TAIL)])

    return kern


def _make_scatter(w):
    scratch = [pltpu.VMEM((_CH,), jnp.int32),
               pltpu.VMEM((_CH, w), F32),
               pltpu.SemaphoreType.DMA,
               pltpu.VMEM((_CH,), jnp.int32),
               pltpu.VMEM((_CH, w), F32),
               pltpu.SemaphoreType.DMA,
               pltpu.VMEM((_TAIL,), jnp.int32),
               pltpu.VMEM((_TAIL, w), F32),
               pltpu.VMEM_SHARED((N, w), F32)]

    @functools.partial(pl.kernel,
                       out_type=jax.ShapeDtypeStruct((2, N, w), F32),
                       mesh=_sc_mesh(), scratch_types=scratch,
                       compiler_params=_SC_PARAMS)
    def kern(vals, dsti, zrows, out, iv0, rv0, sem0, iv1, rv1, sem1,
             ivt, rvt, acc):
        cid = jax.lax.axis_index("c")
        sid = jax.lax.axis_index("s")
        base0 = (sid * 2 + cid) * _PW
        stripe = pl.ds(sid * _SUBROWS, _SUBROWS)
        pltpu.sync_copy(zrows.at[stripe], acc.at[stripe])
        plsc.subcore_barrier()

        bufs = ((iv0, rv0, sem0), (iv1, rv1, sem1))

        def start(c, p):
            iv, rv, sem = bufs[p]
            base = base0 + c * _CH
            pltpu.sync_copy(dsti.at[pl.ds(base, _CH)], iv)
            pltpu.async_copy(vals.at[pl.ds(base, _CH)], rv, sem)

        def finish(c, p):
            iv, rv, sem = bufs[p]
            base = base0 + c * _CH
            pltpu.make_async_copy(vals.at[pl.ds(base, _CH)], rv, sem).wait()
            pltpu.sync_copy(rv, acc.at[iv], add=True)

        start(0, 0)
        if _NFULL > 1:
            start(1, 1)

        @pl.loop(0, (_NFULL + 1) // 2)
        def _(i):
            c0 = 2 * i
            finish(c0, 0)

            @pl.when(c0 + 2 < _NFULL)
            def _():
                start(c0 + 2, 0)

            @pl.when(c0 + 1 < _NFULL)
            def _():
                finish(c0 + 1, 1)

            @pl.when(c0 + 3 < _NFULL)
            def _():
                start(c0 + 3, 1)

        if _TAIL:
            base = base0 + _NFULL * _CH
            pltpu.sync_copy(dsti.at[pl.ds(base, _TAIL)], ivt)
            pltpu.sync_copy(vals.at[pl.ds(base, _TAIL)], rvt)
            pltpu.sync_copy(rvt, acc.at[ivt], add=True)

        plsc.subcore_barrier()
        pltpu.sync_copy(acc.at[stripe], out.at[cid, stripe])

    return kern


_GATHER_PPXX = _make_gather((HID, HID, 16, 16))
_GATHER_XXXX = _make_gather((16, 16, 16, 16))
_GATHER_QQ = _make_gather((HID, HID))
_SCATTER_MSG = _make_scatter(MSG)
_SCATTER_POS = _make_scatter(16)


def _gather4(t0, i0, t1, i1, t2, i2, t3, i3):
    if t0.shape[1] == HID:
        return _GATHER_PPXX(t0, t1, t2, t3, i0, i1, i2, i3)
    return _GATHER_XXXX(t0, t1, t2, t3, i0, i1, i2, i3)


def _gather2(t0, i0, t1, i1):
    return _GATHER_QQ(t0, t1, i0, i1)


def _scatter_planes(vals, dst, width):
    zrows = jnp.zeros((N, width), F32)
    if width == MSG:
        return _SCATTER_MSG(vals, dst, zrows)
    return _SCATTER_POS(vals, dst, zrows)


def _split_edge_w1(w1):
    wef = w1[2 * SS:]
    ws40 = jnp.concatenate(
        [wef[0:2], wef[4:7], jnp.zeros((3, HID), F32),
         wef[17:27], jnp.zeros((6, HID), F32),
         wef[27:37], jnp.zeros((6, HID), F32)], axis=0)
    wr16 = jnp.concatenate([wef[7:17], jnp.zeros((6, HID), F32)], axis=0)
    return (_pad_rows(w1[:SS], SSP), _pad_rows(w1[SS:2 * SS], SSP),
            ws40, wr16, wef[2:3], wef[3:4])


def kernel(xA_x, xB_x, s, is_bond_A, is_bond_B, params, Z, edge_index):
    s = s.reshape(-1)
    src = edge_index[0].astype(jnp.int32)
    dst = edge_index[1].astype(jnp.int32)

    z2 = Z.astype(jnp.int32).reshape(N, 1)
    xa16 = _pad_cols(xA_x, 16)
    xb16 = _pad_cols(xB_x, 16)
    s8 = _pad_cols(s.reshape(N, 1), 8)
    ib2 = _pad_cols(jnp.stack([is_bond_A, is_bond_B], axis=1), 8)

    p = params
    (wat0, bat0), (wat1, bat1) = p["atom_info"]
    (wa, ba), = p["WA"]
    (wb, bb), = p["WB"]
    waz, wax = wa[:NZ], _pad_rows(wa[NZ:], 16)
    wbz, wbx = wb[:NZ], _pad_rows(wb[NZ:], 16)

    msg_w = []
    for l in range(NL):
        (w1, b1), (w2, b2), (w3, b3) = p["msg"][l]
        wsrc, wdst, ws40, wr16, wd, wd2 = _split_edge_w1(w1)
        msg_w.append((wsrc, b1.reshape(1, -1), wdst, ws40, wr16, wd, wd2,
                      w2, b2.reshape(1, -1), w3, b3.reshape(1, -1)))
    alpha_w = []
    for l in range(NL):
        (w1, b1), (w2, b2), (w3, b3) = p["alpha"][l]
        wsrc, wdst, ws40, wr16, wd, wd2 = _split_edge_w1(w1)
        alpha_w.append((wsrc, b1.reshape(1, -1), wdst, ws40, wr16, wd, wd2,
                        w2, b2.reshape(1, -1), _pad_cols(w3, 8),
                        _pad_cols(b3.reshape(1, 1), 8)))
    node_w = []
    for l in range(NL):
        (sw1, sb1), (sw2, sb2), (sw3, sb3) = p["state"][l]
        (bw1, bb1), (bw2, bb2), (bw3, bb3) = p["beta"][l]
        (gw1, gb1), (gw2, gb2), (gw3, gb3) = p["gamma"][l]
        if l + 1 < NL:
            wns, bns, wnd = msg_w[l + 1][0], msg_w[l + 1][1], msg_w[l + 1][2]
        else:
            wns = jnp.zeros((SSP, HID), F32)
            bns = jnp.zeros((1, HID), F32)
            wnd = jnp.zeros((SSP, HID), F32)
        aws, ab1, awd = alpha_w[l][0], alpha_w[l][1], alpha_w[l][2]
        gw3p = jnp.concatenate(
            [jnp.zeros((HID, 1), F32), gw3, jnp.zeros((HID, 6), F32)], axis=1)
        gb3p = jnp.concatenate(
            [jnp.zeros((1, 1), F32), gb3.reshape(1, 1), jnp.zeros((1, 6), F32)], axis=1)
        node_w.append((
            _pad_rows(sw1[:SS], SSP), sw1[SS:], sb1.reshape(1, -1),
            sw2, sb2.reshape(1, -1), _pad_cols(sw3, SSP),
            _pad_cols(sb3.reshape(1, -1), SSP),
            aws, ab1, awd,
            _pad_rows(bw1, SSP), bb1.reshape(1, -1), bw2, bb2.reshape(1, -1),
            _pad_cols(bw3, 8), _pad_cols(bb3.reshape(1, 1), 8),
            _pad_rows(gw1, SSP), gb1.reshape(1, -1), gw2, gb2.reshape(1, -1),
            gw3p, gb3p,
            wns, bns, wnd))

    h, x16, ps, pd = _node_pre(
        z2, xa16, xb16, s8, wat0, bat0.reshape(1, -1), wat1, bat1.reshape(1, -1),
        waz, wax, ba.reshape(1, -1), wbz, wbx, bb.reshape(1, -1),
        msg_w[0][0], msg_w[0][1], msg_w[0][2])

    xas, xad, xbs, xbd = _gather4(xa16, src, xa16, dst, xb16, src, xb16, dst)
    st40 = _edge_static(xas, xad, xbs, xbd, ib2)

    for l in range(NL):
        last = (l + 1 == NL)
        mw = msg_w[l]
        aw = alpha_w[l]
        psg, pdg, xs, xd = _gather4(ps, src, pd, dst, x16, src, x16, dst)
        messages = _edge_msg(psg, pdg, xs, xd, st40,
                             mw[3], mw[4], mw[5], mw[6], mw[7], mw[8], mw[9], mw[10])
        nm = _scatter_planes(messages, dst, MSG)
        h, qs, qd, bg8, ps, pd = _node_state(h, nm[0], nm[1], s8, node_w[l], last)
        qsg, qdg = _gather2(qs, src, qd, dst)
        av = _edge_alpha(qsg, qdg, xs, xd, st40,
                         aw[3], aw[4], aw[5], aw[6], aw[7], aw[8], aw[9], aw[10])
        nu = _scatter_planes(av, dst, 16)
        x16 = _x_update(x16, nu[0], nu[1], bg8, xa16, xb16, last)

    return x16[:, :3]
